# baseline (speedup 1.0000x reference)
import functools

import jax
import jax.numpy as jnp
from jax import lax
from jax.experimental import pallas as pl
from jax.experimental.pallas import tpu as pltpu
from jax.experimental.pallas import tpu_sc as plsc

N = 10000
D = 256
DH = 128
NLAYERS = 6
NC = 2
NS = 16
CH = 128
E_RAW = 160000 + N
E_PAD = ((E_RAW + NS * CH - 1) // (NS * CH)) * (NS * CH)
N_PAD = 10240
ROWS_T = N_PAD // NS
BN = 1000


NB = 2
SUP = 2 * NB
NCHUNK = E_PAD // (NS * CH)
NSUPER = NCHUNK // SUP


def _make_sc_pass(gather: bool, dw: int = DH):
    mesh = plsc.VectorSubcoreMesh(core_axis_name="c", subcore_axis_name="s")

    @functools.partial(
        pl.kernel,
        mesh=mesh,
        out_type=jax.ShapeDtypeStruct((NC, N_PAD, dw), jnp.float32),
        scratch_types=[pltpu.VMEM((SUP, CH), jnp.int32)] * 2
        + [pltpu.VMEM((SUP, CH), jnp.int32)] * 2
        + [pltpu.VMEM((CH, dw), jnp.float32)] * NB
        + [pltpu.SemaphoreType.DMA] * 2
        + [pltpu.SemaphoreType.DMA] * NB
        + [pltpu.SemaphoreType.DMA] * NB
        + [pltpu.VMEM_SHARED((N_PAD, dw), jnp.float32)],
    )
    def sc_pass(tbl, gidx, ridx, fill, out, *rest):
        gi = rest[:2]
        ri = rest[2:4]
        rows = rest[4:4 + NB]
        isems = rest[4 + NB:6 + NB]
        gsems = rest[6 + NB:6 + 2 * NB]
        ssems = rest[6 + 2 * NB:6 + 3 * NB]
        acc = rest[-1]
        c = lax.axis_index("c")
        s = lax.axis_index("s")

        def load_idx(sup, slot):
            pltpu.async_copy(ridx.at[s, sup], ri[slot], isems[slot])
            if gather:
                pltpu.async_copy(gidx.at[c, s, sup], gi[slot], isems[slot])

        def wait_idx(slot):
            pltpu.make_async_copy(ridx.at[s, 0], ri[slot],
                                  isems[slot]).wait()
            if gather:
                pltpu.make_async_copy(ridx.at[s, 0], gi[slot],
                                      isems[slot]).wait()

        def drain_scatter(b):
            pltpu.make_async_copy(tbl.at[pl.ds(0, CH)], rows[b],
                                  ssems[b]).wait()

        load_idx(0, 0)
        pltpu.sync_copy(fill.at[0], rows[0])
        zcopies = [
            pltpu.async_copy(rows[0], acc.at[pl.ds(s * ROWS_T + j * CH, CH)],
                             gsems[0])
            for j in range(ROWS_T // CH)
        ]
        for zc in zcopies:
            zc.wait()
        if not gather:
            for b in range(NB):
                pltpu.sync_copy(fill.at[1], rows[b])
        plsc.subcore_barrier()

        def do_super(slot, drain_pred, pre_sup, pre_slot):
            wait_idx(slot)
            for p in range(2):
                for b in range(NB):
                    if p == 0 and drain_pred is not None:
                        @pl.when(drain_pred)
                        def _(b=b):
                            drain_scatter(b)
                    else:
                        drain_scatter(b)
                    if gather:
                        pltpu.async_copy(tbl.at[gi[slot].at[p * NB + b]],
                                         rows[b], gsems[b])
                for b in range(NB):
                    if gather:
                        pltpu.make_async_copy(tbl.at[pl.ds(0, CH)], rows[b],
                                              gsems[b]).wait()
                    pltpu.async_copy(rows[b], acc.at[ri[slot].at[p * NB + b]],
                                     ssems[b], add=True)
                if p == 0 and pre_sup is not None:
                    load_idx(pre_sup, pre_slot)

        def pair_(m, carry):
            do_super(0, m > 0, 2 * m + 1, 1)
            do_super(1, None, 2 * m + 2, 0)
            return carry

        lax.fori_loop(0, NSUPER // 2, pair_, 0)
        do_super(0, None, None, None)
        for b in range(NB):
            drain_scatter(b)
        plsc.subcore_barrier()
        nwb = ROWS_T // CH
        rd, wr = {}, {}
        for j in range(nwb):
            b = j % NB
            if j >= NB:
                wr[j - NB].wait()
            start = s * ROWS_T + j * CH
            rd[j] = pltpu.async_copy(acc.at[pl.ds(start, CH)], rows[b],
                                     gsems[b])
            rd[j].wait()
            wr[j] = pltpu.async_copy(rows[b], out.at[c, pl.ds(start, CH)],
                                     ssems[b])
        for j in range(max(0, nwb - NB), nwb):
            wr[j].wait()

    return sc_pass


DW = DH
_sc_spmm = _make_sc_pass(gather=True)
_sc_deg = _make_sc_pass(gather=False, dw=DW)


def _tc_mm0(x_ref, w0_ref, t_ref):
    t_ref[...] = jnp.dot(x_ref[...], w0_ref[...],
                         preferred_element_type=jnp.float32)


def _tc_scale0(deg_ref, t_ref, dis_ref, t0_ref):
    dis = lax.rsqrt(deg_ref[0])
    dis_ref[...] = dis
    d2 = jnp.concatenate([dis, dis], axis=1)
    t0_ref[...] = t_ref[...] * d2


def _tc_layer(u_ref, dis_ref, b_ref, w_ref, h_ref, t_ref):
    dis = dis_ref[...]
    d2 = jnp.concatenate([dis, dis], axis=1)
    u = jnp.concatenate([u_ref[0], u_ref[1]], axis=1)
    h = jnp.maximum(u * d2 + b_ref[...], 0.0)
    h_ref[...] = h
    t = jnp.dot(h, w_ref[...], preferred_element_type=jnp.float32)
    t_ref[...] = t * d2


def _tc_jk(h_ref, lw_ref, pacc_ref, out_ref):
    out_ref[...] = pacc_ref[...] + jnp.dot(
        h_ref[...], lw_ref[...], preferred_element_type=jnp.float32)


def _tc_epilogue(u_ref, dis_ref, b_ref, pacc_ref, lw_ref, lb_ref, out_ref):
    dis = dis_ref[...]
    d2 = jnp.concatenate([dis, dis], axis=1)
    u = jnp.concatenate([u_ref[0], u_ref[1]], axis=1)
    h5 = jnp.maximum(u * d2 + b_ref[...], 0.0)
    acc = pacc_ref[...] + lb_ref[...] + jnp.dot(
        h5, lw_ref[...], preferred_element_type=jnp.float32)
    m = jnp.max(acc, axis=1, keepdims=True)
    e = jnp.exp(acc - m)
    out_ref[...] = acc - m - jnp.log(jnp.sum(e, axis=1, keepdims=True))


def _row_block(d):
    return pl.BlockSpec((BN, d), lambda i: (i, 0))


def _half_block():
    return pl.BlockSpec((NC, BN, DH), lambda i: (0, i, 0))


def _full_block(r, c):
    return pl.BlockSpec((r, c), lambda i: (0, 0))


def kernel(x, edge_index, convW, convB, lin_W, lin_b):
    row = edge_index[1].astype(jnp.int32)
    col = edge_index[0].astype(jnp.int32)
    loop = jnp.arange(N, dtype=jnp.int32)
    row = jnp.concatenate([row, loop])
    col = jnp.concatenate([col, loop])
    npad = E_PAD - E_RAW
    row = jnp.concatenate([row, jnp.full((npad,), N, jnp.int32)])
    col = jnp.concatenate([col, jnp.zeros((npad,), jnp.int32)])
    row = row.reshape(NS, NSUPER, SUP, CH)
    col = col.reshape(NS, NSUPER, SUP, CH)
    gidx = jnp.stack([2 * col, 2 * col + 1])
    fill = jnp.stack([jnp.zeros((CH, DH), jnp.float32),
                      jnp.ones((CH, DH), jnp.float32)])
    fill_deg = jnp.stack([jnp.zeros((CH, DW), jnp.float32),
                          jnp.ones((CH, DW), jnp.float32)])
    dummy_tbl = jnp.zeros((2 * N, DW), jnp.float32)

    deg = _sc_deg(dummy_tbl, gidx, row, fill_deg)

    grid = (N // BN,)
    mm0 = pl.pallas_call(
        _tc_mm0,
        grid=grid,
        in_specs=[_row_block(D), _full_block(D, D)],
        out_specs=_row_block(D),
        out_shape=jax.ShapeDtypeStruct((N, D), jnp.float32),
    )(x, convW[0])
    dis, t = pl.pallas_call(
        _tc_scale0,
        grid=grid,
        in_specs=[pl.BlockSpec((NC, BN, DW), lambda i: (0, i, 0)),
                  _row_block(D)],
        out_specs=[_row_block(DH), _row_block(D)],
        out_shape=[jax.ShapeDtypeStruct((N, DH), jnp.float32),
                   jax.ShapeDtypeStruct((N, D), jnp.float32)],
    )(deg, mm0)

    pacc = jnp.zeros((N, D), jnp.float32)
    for i in range(NLAYERS):
        u = _sc_spmm(t.reshape(2 * N, DH), gidx, row, fill)
        if i == NLAYERS - 1:
            break
        h, t = pl.pallas_call(
            _tc_layer,
            grid=grid,
            in_specs=[_half_block(), _row_block(DH), _full_block(1, D),
                      _full_block(D, D)],
            out_specs=[_row_block(D), _row_block(D)],
            out_shape=[jax.ShapeDtypeStruct((N, D), jnp.float32),
                       jax.ShapeDtypeStruct((N, D), jnp.float32)],
        )(u, dis, convB[i].reshape(1, D), convW[i + 1])
        pacc = pl.pallas_call(
            _tc_jk,
            grid=grid,
            in_specs=[_row_block(D), _full_block(D, D), _row_block(D)],
            out_specs=_row_block(D),
            out_shape=jax.ShapeDtypeStruct((N, D), jnp.float32),
        )(h, lax.dynamic_slice_in_dim(lin_W, i * D, D), pacc)

    out = pl.pallas_call(
        _tc_epilogue,
        grid=grid,
        in_specs=[_half_block(), _row_block(DH), _full_block(1, D),
                  _row_block(D), _full_block(D, D), _full_block(1, D)],
        out_specs=_row_block(D),
        out_shape=jax.ShapeDtypeStruct((N, D), jnp.float32),
    )(u, dis, convB[NLAYERS - 1].reshape(1, D), pacc,
      lin_W[(NLAYERS - 1) * D:], lin_b.reshape(1, D))
    return out
